# baseline (speedup 1.0000x reference)
import functools

import jax
import jax.numpy as jnp
from jax import lax
from jax.experimental import pallas as pl
from jax.experimental.pallas import tpu as pltpu
from jax.experimental.pallas import tpu_sc as plsc

N = 4096
D_IN = 1024
D_H = 512
D_QK = 256
NNB = 9

NC = 2
NS = 16
LANES = 16
NW = NC * NS
ROWS_PER_W = N // NW
CHUNK = 16
NCHUNKS = ROWS_PER_W // CHUNK
NPAIR = D_QK // (2 * LANES)
DW = D_QK // 2


def _proj_body(x_ref, w1_ref, b1_ref, wq_ref, wk_ref, wv_ref, bv_ref,
               q_ref, k_ref, v_ref):
    bf = jnp.bfloat16
    dense = jnp.dot(x_ref[...].astype(bf), w1_ref[...].astype(bf),
                    preferred_element_type=jnp.float32)
    dense = jnp.maximum(dense + b1_ref[...], 0.0).astype(bf)
    q_ref[...] = (jnp.dot(dense, wq_ref[...].astype(bf),
                          preferred_element_type=jnp.float32)
                  * (1.0 / 16.0)).astype(bf)
    k_ref[...] = jnp.dot(dense, wk_ref[...].astype(bf),
                         preferred_element_type=jnp.float32).astype(bf)
    v_ref[...] = jnp.dot(dense, wv_ref[...].astype(bf),
                         preferred_element_type=jnp.float32) + bv_ref[...]


_PROJ_TILE = 512


def _projections(x, W1, b1, Wq, Wk, Wv, bv):
    grid = (N // _PROJ_TILE,)
    out_shape = [
        jax.ShapeDtypeStruct((N, D_QK), jnp.bfloat16),
        jax.ShapeDtypeStruct((N, D_QK), jnp.bfloat16),
        jax.ShapeDtypeStruct((N, D_QK), jnp.float32),
    ]
    return pl.pallas_call(
        _proj_body,
        grid=grid,
        in_specs=[
            pl.BlockSpec((_PROJ_TILE, D_IN), lambda i: (i, 0)),
            pl.BlockSpec((D_IN, D_H), lambda i: (0, 0)),
            pl.BlockSpec((1, D_H), lambda i: (0, 0)),
            pl.BlockSpec((D_H, D_QK), lambda i: (0, 0)),
            pl.BlockSpec((D_H, D_QK), lambda i: (0, 0)),
            pl.BlockSpec((D_H, D_QK), lambda i: (0, 0)),
            pl.BlockSpec((1, D_QK), lambda i: (0, 0)),
        ],
        out_specs=[pl.BlockSpec((_PROJ_TILE, D_QK), lambda i: (i, 0))] * 3,
        out_shape=out_shape,
    )(x, W1, b1, Wq, Wk, Wv, bv)


NSLOT = 16
SUB = 128 // NSLOT


def _sc_body(t_hbm, idx2_hbm, out_hbm, idx_v, g0_v, g1_v, res_v,
             isem, sem0, sem1):
    wid = lax.axis_index("s") * NC + lax.axis_index("c")
    row0 = wid * ROWS_PER_W
    lane = jnp.arange(LANES, dtype=jnp.int32)
    pltpu.async_copy(idx2_hbm.at[wid * (2 * NCHUNKS) + lane], idx_v,
                     isem).wait()

    gbufs = (g0_v, g1_v)
    sems = (sem0, sem1)

    def _gather(ci, buf, sem):
        return [
            pltpu.make_async_copy(
                t_hbm.at[idx_v.at[2 * ci + h]],
                buf.at[pl.ds(h * SUB * NSLOT, SUB * NSLOT)], sem)
            for h in range(2)
        ]

    for cp in _gather(0, g0_v, sem0):
        cp.start()

    def compute_chunk(ci, g_v):
        def row_body(r, carry):
            v3, v5, v9 = carry
            g0 = r * NSLOT
            t3 = jnp.zeros((LANES,), jnp.float32)
            t5 = jnp.zeros((LANES,), jnp.float32)
            t9 = jnp.zeros((LANES,), jnp.float32)
            for c in range(NPAIR):
                sl = pl.ds(c * LANES, LANES)
                gld = lambda j: plsc.bitcast(g_v[g0 + 1 + j, sl],
                                             jnp.bfloat16)
                qa, qb = plsc.unpack(
                    plsc.bitcast(g_v[g0, sl], jnp.bfloat16),
                    format=plsc.PackFormat.INTERLEAVED)
                s = gld(0) + gld(1) + gld(2)
                a, b = plsc.unpack(s, format=plsc.PackFormat.INTERLEAVED)
                t3 = t3 + qa * a + qb * b
                s = s + gld(3) + gld(4)
                a, b = plsc.unpack(s, format=plsc.PackFormat.INTERLEAVED)
                t5 = t5 + qa * a + qb * b
                s = s + gld(5) + gld(6) + gld(7) + gld(8)
                a, b = plsc.unpack(s, format=plsc.PackFormat.INTERLEAVED)
                t9 = t9 + qa * a + qb * b
            m = lane == r
            v3 = jnp.where(m, jnp.sum(t3), v3)
            v5 = jnp.where(m, jnp.sum(t5), v5)
            v9 = jnp.where(m, jnp.sum(t9), v9)
            return v3, v5, v9

        zero = jnp.zeros((LANES,), jnp.float32)
        v3, v5, v9 = lax.fori_loop(0, CHUNK, row_body, (zero, zero, zero))
        res_v[0, pl.ds(ci * CHUNK, CHUNK)] = v3
        res_v[1, pl.ds(ci * CHUNK, CHUNK)] = v5
        res_v[2, pl.ds(ci * CHUNK, CHUNK)] = v9

    def super_body(s, _):
        for b in range(2):
            ci = 2 * s + b

            @pl.when(ci + 1 < NCHUNKS)
            def _():
                for cp in _gather(ci + 1, gbufs[1 - b], sems[1 - b]):
                    cp.start()

            for cp in _gather(ci, gbufs[b], sems[b]):
                cp.wait()
            compute_chunk(ci, gbufs[b])
        return 0

    lax.fori_loop(0, NCHUNKS // 2, super_body, 0)
    pltpu.sync_copy(res_v, out_hbm.at[:, pl.ds(row0, ROWS_PER_W)])


def _sc_gather_dot(table, idx2):
    mesh = plsc.VectorSubcoreMesh(core_axis_name="c", subcore_axis_name="s",
                                  num_cores=NC, num_subcores=NS)
    kern = functools.partial(
        pl.kernel,
        out_type=jax.ShapeDtypeStruct((3, N), jnp.float32),
        mesh=mesh,
        compiler_params=pltpu.CompilerParams(needs_layout_passes=False),
        scratch_types=[
            pltpu.VMEM((LANES, SUB * NSLOT), jnp.int32),
            pltpu.VMEM((CHUNK * NSLOT, DW), jnp.int32),
            pltpu.VMEM((CHUNK * NSLOT, DW), jnp.int32),
            pltpu.VMEM((3, ROWS_PER_W), jnp.float32),
            pltpu.SemaphoreType.DMA,
            pltpu.SemaphoreType.DMA,
            pltpu.SemaphoreType.DMA,
        ],
    )(_sc_body)
    return kern(table, idx2)


def _pool_body(a_ref, v_ref, wout_ref, bout_ref, o_ref):
    A = a_ref[...]
    m = jnp.max(A, axis=1, keepdims=True)
    e = jnp.exp(A - m)
    alpha = e / jnp.sum(e, axis=1, keepdims=True)
    s3 = jnp.dot(alpha, v_ref[...], preferred_element_type=jnp.float32)
    s = jnp.sum(s3, axis=0, keepdims=True)
    logits = jnp.dot(s, wout_ref[...],
                     preferred_element_type=jnp.float32) + bout_ref[...]
    ls = logits - jnp.max(logits, axis=1, keepdims=True)
    el = jnp.exp(ls)
    o_ref[...] = el / jnp.sum(el, axis=1, keepdims=True)


def _pool(A, value, Wout, bout):
    return pl.pallas_call(
        _pool_body,
        out_shape=jax.ShapeDtypeStruct((1, 2), jnp.float32),
    )(A, value, Wout, bout)


def kernel(x, neighbor_idx, W1, b1, Wq, Wk, Wv, bv, Wout, bout):
    q, kmat, value = _projections(x, W1, b1.reshape(1, -1), Wq, Wk, Wv,
                                  bv.reshape(1, -1))
    rows = jnp.arange(N, dtype=jnp.int32)[:, None]
    slots = jnp.concatenate(
        [rows, neighbor_idx.astype(jnp.int32) + N,
         jnp.broadcast_to(rows, (N, NSLOT - 1 - NNB))], axis=1)
    idx2 = slots.reshape(NW * 2 * NCHUNKS, SUB * NSLOT)
    table = jax.lax.bitcast_convert_type(
        jnp.concatenate([q, kmat], axis=0).reshape(2 * N, DW, 2), jnp.int32)
    A = _sc_gather_dot(table, idx2)
    return _pool(A, value, Wout, bout.reshape(1, -1))


# speedup vs baseline: 1.0737x; 1.0737x over previous
import functools

import jax
import jax.numpy as jnp
from jax import lax
from jax.experimental import pallas as pl
from jax.experimental.pallas import tpu as pltpu
from jax.experimental.pallas import tpu_sc as plsc

N = 4096
D_IN = 1024
D_H = 512
D_QK = 256
NNB = 9

NC = 2
NS = 16
LANES = 16
NW = NC * NS
ROWS_PER_W = N // NW
CHUNK = 16
NCHUNKS = ROWS_PER_W // CHUNK
NPAIR = D_QK // (2 * LANES)
DW = D_QK // 2


def _proj_body(x_ref, w1_ref, b1_ref, wq_ref, wk_ref, wv_ref, bv_ref,
               q_ref, k_ref, v_ref):
    bf = jnp.bfloat16
    dense = jnp.dot(x_ref[...].astype(bf), w1_ref[...].astype(bf),
                    preferred_element_type=jnp.float32)
    dense = jnp.maximum(dense + b1_ref[...], 0.0).astype(bf)
    q_ref[...] = (jnp.dot(dense, wq_ref[...].astype(bf),
                          preferred_element_type=jnp.float32)
                  * (1.0 / 16.0)).astype(bf)
    k_ref[...] = jnp.dot(dense, wk_ref[...].astype(bf),
                         preferred_element_type=jnp.float32).astype(bf)
    v_ref[...] = jnp.dot(dense, wv_ref[...].astype(bf),
                         preferred_element_type=jnp.float32) + bv_ref[...]


_PROJ_TILE = 512


def _projections(x, W1, b1, Wq, Wk, Wv, bv):
    grid = (N // _PROJ_TILE,)
    out_shape = [
        jax.ShapeDtypeStruct((N, D_QK), jnp.bfloat16),
        jax.ShapeDtypeStruct((N, D_QK), jnp.bfloat16),
        jax.ShapeDtypeStruct((N, D_QK), jnp.float32),
    ]
    return pl.pallas_call(
        _proj_body,
        grid=grid,
        in_specs=[
            pl.BlockSpec((_PROJ_TILE, D_IN), lambda i: (i, 0)),
            pl.BlockSpec((D_IN, D_H), lambda i: (0, 0)),
            pl.BlockSpec((1, D_H), lambda i: (0, 0)),
            pl.BlockSpec((D_H, D_QK), lambda i: (0, 0)),
            pl.BlockSpec((D_H, D_QK), lambda i: (0, 0)),
            pl.BlockSpec((D_H, D_QK), lambda i: (0, 0)),
            pl.BlockSpec((1, D_QK), lambda i: (0, 0)),
        ],
        out_specs=[pl.BlockSpec((_PROJ_TILE, D_QK), lambda i: (i, 0))] * 3,
        out_shape=out_shape,
    )(x, W1, b1, Wq, Wk, Wv, bv)


NSLOT = 10


def _sc_body(t_hbm, idx2_hbm, out_hbm, idx_v, g0_v, g1_v, res_v,
             sem0, sem1):
    wid = lax.axis_index("s") * NC + lax.axis_index("c")
    row0 = wid * ROWS_PER_W
    lane = jnp.arange(LANES, dtype=jnp.int32)
    pltpu.sync_copy(
        idx2_hbm.at[pl.ds(row0 * NSLOT, ROWS_PER_W * NSLOT)], idx_v)

    gbufs = (g0_v, g1_v)
    sems = (sem0, sem1)

    def _gather(ci, buf, sem):
        return pltpu.make_async_copy(
            t_hbm.at[idx_v.at[pl.ds(ci * CHUNK * NSLOT, CHUNK * NSLOT)]],
            buf, sem)

    _gather(0, g0_v, sem0).start()

    def compute_chunk(ci, g_v):
        def row_body(r, carry):
            v3, v5, v9 = carry
            g0 = r * NSLOT
            t3 = jnp.zeros((LANES,), jnp.float32)
            t5 = jnp.zeros((LANES,), jnp.float32)
            t9 = jnp.zeros((LANES,), jnp.float32)
            for c in range(NPAIR):
                sl = pl.ds(c * LANES, LANES)
                gld = lambda j: plsc.bitcast(g_v[g0 + 1 + j, sl],
                                             jnp.bfloat16)
                qa, qb = plsc.unpack(
                    plsc.bitcast(g_v[g0, sl], jnp.bfloat16),
                    format=plsc.PackFormat.INTERLEAVED)
                s = gld(0) + gld(1) + gld(2)
                a, b = plsc.unpack(s, format=plsc.PackFormat.INTERLEAVED)
                t3 = t3 + qa * a + qb * b
                s = s + gld(3) + gld(4)
                a, b = plsc.unpack(s, format=plsc.PackFormat.INTERLEAVED)
                t5 = t5 + qa * a + qb * b
                s = s + gld(5) + gld(6) + gld(7) + gld(8)
                a, b = plsc.unpack(s, format=plsc.PackFormat.INTERLEAVED)
                t9 = t9 + qa * a + qb * b
            m = lane == r
            v3 = jnp.where(m, jnp.sum(t3), v3)
            v5 = jnp.where(m, jnp.sum(t5), v5)
            v9 = jnp.where(m, jnp.sum(t9), v9)
            return v3, v5, v9

        zero = jnp.zeros((LANES,), jnp.float32)
        v3, v5, v9 = lax.fori_loop(0, CHUNK, row_body, (zero, zero, zero))
        res_v[0, pl.ds(ci * CHUNK, CHUNK)] = v3
        res_v[1, pl.ds(ci * CHUNK, CHUNK)] = v5
        res_v[2, pl.ds(ci * CHUNK, CHUNK)] = v9

    def super_body(s, _):
        for b in range(2):
            ci = 2 * s + b

            @pl.when(ci + 1 < NCHUNKS)
            def _():
                _gather(ci + 1, gbufs[1 - b], sems[1 - b]).start()

            _gather(ci, gbufs[b], sems[b]).wait()
            compute_chunk(ci, gbufs[b])
        return 0

    lax.fori_loop(0, NCHUNKS // 2, super_body, 0)
    pltpu.sync_copy(res_v, out_hbm.at[:, pl.ds(row0, ROWS_PER_W)])


def _sc_gather_dot(table, idx2):
    mesh = plsc.VectorSubcoreMesh(core_axis_name="c", subcore_axis_name="s",
                                  num_cores=NC, num_subcores=NS)
    kern = functools.partial(
        pl.kernel,
        out_type=jax.ShapeDtypeStruct((3, N), jnp.float32),
        mesh=mesh,
        compiler_params=pltpu.CompilerParams(needs_layout_passes=False),
        scratch_types=[
            pltpu.VMEM((ROWS_PER_W * NSLOT,), jnp.int32),
            pltpu.VMEM((CHUNK * NSLOT, DW), jnp.int32),
            pltpu.VMEM((CHUNK * NSLOT, DW), jnp.int32),
            pltpu.VMEM((3, ROWS_PER_W), jnp.float32),
            pltpu.SemaphoreType.DMA,
            pltpu.SemaphoreType.DMA,
        ],
    )(_sc_body)
    return kern(table, idx2)


def _pool_body(a_ref, v_ref, wout_ref, bout_ref, o_ref):
    A = a_ref[...]
    m = jnp.max(A, axis=1, keepdims=True)
    e = jnp.exp(A - m)
    alpha = e / jnp.sum(e, axis=1, keepdims=True)
    s3 = jnp.dot(alpha, v_ref[...], preferred_element_type=jnp.float32)
    s = jnp.sum(s3, axis=0, keepdims=True)
    logits = jnp.dot(s, wout_ref[...],
                     preferred_element_type=jnp.float32) + bout_ref[...]
    ls = logits - jnp.max(logits, axis=1, keepdims=True)
    el = jnp.exp(ls)
    o_ref[...] = el / jnp.sum(el, axis=1, keepdims=True)


def _pool(A, value, Wout, bout):
    return pl.pallas_call(
        _pool_body,
        out_shape=jax.ShapeDtypeStruct((1, 2), jnp.float32),
    )(A, value, Wout, bout)


def kernel(x, neighbor_idx, W1, b1, Wq, Wk, Wv, bv, Wout, bout):
    q, kmat, value = _projections(x, W1, b1.reshape(1, -1), Wq, Wk, Wv,
                                  bv.reshape(1, -1))
    rows = jnp.arange(N, dtype=jnp.int32)[:, None]
    slots = jnp.concatenate(
        [rows, neighbor_idx.astype(jnp.int32) + N], axis=1)
    idx2 = slots.reshape(-1)
    table = jax.lax.bitcast_convert_type(
        jnp.concatenate([q, kmat], axis=0).reshape(2 * N, DW, 2), jnp.int32)
    A = _sc_gather_dot(table, idx2)
    return _pool(A, value, Wout, bout.reshape(1, -1))


# speedup vs baseline: 1.3294x; 1.2381x over previous
import functools

import jax
import jax.numpy as jnp
import numpy as np
from jax import lax
from jax.experimental import pallas as pl
from jax.experimental.pallas import tpu as pltpu
from jax.experimental.pallas import tpu_sc as plsc

N = 4096
D_IN = 1024
D_H = 512
D_QK = 256
NNB = 9

NC = 2
NS = 16
LANES = 16
NW = NC * NS
ROWS_PER_W = N // NW
CHUNK = 16
NCHUNKS = ROWS_PER_W // CHUNK
NPAIR = D_QK // (2 * LANES)

_PERM = np.arange(D_QK).reshape(NPAIR, LANES, 2).transpose(0, 2, 1).reshape(-1)


def _proj_body(x_ref, w1_ref, b1_ref, wq_ref, wk_ref, wv_ref, bv_ref,
               q_ref, k_ref, v_ref):
    bf = jnp.bfloat16
    dense = jnp.dot(x_ref[...].astype(bf), w1_ref[...].astype(bf),
                    preferred_element_type=jnp.float32)
    dense = jnp.maximum(dense + b1_ref[...], 0.0).astype(bf)
    q_ref[...] = jnp.dot(dense, wq_ref[...].astype(bf),
                         preferred_element_type=jnp.float32) * (1.0 / 16.0)
    k_ref[...] = jnp.dot(dense, wk_ref[...].astype(bf),
                         preferred_element_type=jnp.float32).astype(bf)
    v_ref[...] = jnp.dot(dense, wv_ref[...].astype(bf),
                         preferred_element_type=jnp.float32) + bv_ref[...]


_PROJ_TILE = 512


def _projections(x, W1, b1, Wq, Wk, Wv, bv):
    grid = (N // _PROJ_TILE,)
    out_shape = [
        jax.ShapeDtypeStruct((N, D_QK), jnp.float32),
        jax.ShapeDtypeStruct((N, D_QK), jnp.bfloat16),
        jax.ShapeDtypeStruct((N, D_QK), jnp.float32),
    ]
    return pl.pallas_call(
        _proj_body,
        grid=grid,
        in_specs=[
            pl.BlockSpec((_PROJ_TILE, D_IN), lambda i: (i, 0)),
            pl.BlockSpec((D_IN, D_H), lambda i: (0, 0)),
            pl.BlockSpec((1, D_H), lambda i: (0, 0)),
            pl.BlockSpec((D_H, D_QK), lambda i: (0, 0)),
            pl.BlockSpec((D_H, D_QK), lambda i: (0, 0)),
            pl.BlockSpec((D_H, D_QK), lambda i: (0, 0)),
            pl.BlockSpec((1, D_QK), lambda i: (0, 0)),
        ],
        out_specs=[pl.BlockSpec((_PROJ_TILE, D_QK), lambda i: (i, 0))] * 3,
        out_shape=out_shape,
    )(x, W1, b1, Wq, Wk, Wv, bv)


def _sc_body(q_hbm, k_hbm, idx_hbm, out_hbm, q_v, idx_v, g0_v, g1_v, res_v,
             sem0, sem1):
    wid = lax.axis_index("s") * NC + lax.axis_index("c")
    row0 = wid * ROWS_PER_W
    pltpu.sync_copy(q_hbm.at[pl.ds(row0, ROWS_PER_W)], q_v)
    pltpu.sync_copy(idx_hbm.at[pl.ds(row0 * NNB, ROWS_PER_W * NNB)], idx_v)

    lane = jnp.arange(LANES, dtype=jnp.int32)
    bufs = (g0_v, g1_v)
    sems = (sem0, sem1)

    def _gather(ci, buf, sem):
        return pltpu.async_copy(
            k_hbm.at[idx_v.at[pl.ds(ci * CHUNK * NNB, CHUNK * NNB)]],
            buf, sem)

    _gather(0, g0_v, sem0)

    def compute_chunk(ci, g_v):
        def row_body(r, carry):
            v3, v5, v9 = carry
            rr = ci * CHUNK + r
            g0 = r * NNB
            t3 = jnp.zeros((LANES,), jnp.float32)
            t5 = jnp.zeros((LANES,), jnp.float32)
            t9 = jnp.zeros((LANES,), jnp.float32)
            for c in range(NPAIR):
                sl = pl.ds(c * LANES, LANES)
                gld = lambda j: plsc.bitcast(g_v[g0 + j, sl], jnp.bfloat16)
                s = gld(0) + gld(1) + gld(2)
                a, b = plsc.unpack(s, format=plsc.PackFormat.INTERLEAVED)
                qa = q_v[rr, pl.ds(c * 2 * LANES, LANES)]
                qb = q_v[rr, pl.ds(c * 2 * LANES + LANES, LANES)]
                t3 = t3 + qa * a + qb * b
                s = s + gld(3) + gld(4)
                a, b = plsc.unpack(s, format=plsc.PackFormat.INTERLEAVED)
                t5 = t5 + qa * a + qb * b
                s = s + gld(5) + gld(6) + gld(7) + gld(8)
                a, b = plsc.unpack(s, format=plsc.PackFormat.INTERLEAVED)
                t9 = t9 + qa * a + qb * b
            m = lane == r
            v3 = jnp.where(m, jnp.sum(t3), v3)
            v5 = jnp.where(m, jnp.sum(t5), v5)
            v9 = jnp.where(m, jnp.sum(t9), v9)
            return v3, v5, v9

        zero = jnp.zeros((LANES,), jnp.float32)
        v3, v5, v9 = lax.fori_loop(0, CHUNK, row_body, (zero, zero, zero))
        res_v[0, pl.ds(ci * CHUNK, CHUNK)] = v3
        res_v[1, pl.ds(ci * CHUNK, CHUNK)] = v5
        res_v[2, pl.ds(ci * CHUNK, CHUNK)] = v9

    def super_body(s, _):
        for b in range(2):
            ci = 2 * s + b

            @pl.when(ci + 1 < NCHUNKS)
            def _():
                _gather(ci + 1, bufs[1 - b], sems[1 - b])

            pltpu.make_async_copy(
                k_hbm.at[idx_v.at[pl.ds(ci * CHUNK * NNB, CHUNK * NNB)]],
                bufs[b], sems[b]).wait()
            compute_chunk(ci, bufs[b])
        return 0

    lax.fori_loop(0, NCHUNKS // 2, super_body, 0)
    pltpu.sync_copy(res_v, out_hbm.at[:, pl.ds(row0, ROWS_PER_W)])


def _sc_gather_dot(q, kmat, nbr_flat):
    mesh = plsc.VectorSubcoreMesh(core_axis_name="c", subcore_axis_name="s",
                                  num_cores=NC, num_subcores=NS)
    kern = functools.partial(
        pl.kernel,
        out_type=jax.ShapeDtypeStruct((3, N), jnp.float32),
        mesh=mesh,
        compiler_params=pltpu.CompilerParams(needs_layout_passes=False),
        scratch_types=[
            pltpu.VMEM((ROWS_PER_W, D_QK), jnp.float32),
            pltpu.VMEM((ROWS_PER_W * NNB,), jnp.int32),
            pltpu.VMEM((CHUNK * NNB, D_QK // 2), jnp.int32),
            pltpu.VMEM((CHUNK * NNB, D_QK // 2), jnp.int32),
            pltpu.VMEM((3, ROWS_PER_W), jnp.float32),
            pltpu.SemaphoreType.DMA,
            pltpu.SemaphoreType.DMA,
        ],
    )(_sc_body)
    return kern(q, kmat, nbr_flat)


def _pool_body(a_ref, v_ref, wout_ref, bout_ref, o_ref):
    A = a_ref[...]
    m = jnp.max(A, axis=1, keepdims=True)
    e = jnp.exp(A - m)
    alpha = e / jnp.sum(e, axis=1, keepdims=True)
    s3 = jnp.dot(alpha, v_ref[...], preferred_element_type=jnp.float32)
    s = jnp.sum(s3, axis=0, keepdims=True)
    logits = jnp.dot(s, wout_ref[...],
                     preferred_element_type=jnp.float32) + bout_ref[...]
    ls = logits - jnp.max(logits, axis=1, keepdims=True)
    el = jnp.exp(ls)
    o_ref[...] = el / jnp.sum(el, axis=1, keepdims=True)


def _pool(A, value, Wout, bout):
    return pl.pallas_call(
        _pool_body,
        out_shape=jax.ShapeDtypeStruct((1, 2), jnp.float32),
    )(A, value, Wout, bout)


def kernel(x, neighbor_idx, W1, b1, Wq, Wk, Wv, bv, Wout, bout):
    q, kmat, value = _projections(x, W1, b1.reshape(1, -1), Wq[:, _PERM],
                                  Wk, Wv, bv.reshape(1, -1))
    nbr_flat = neighbor_idx.astype(jnp.int32).reshape(-1)
    k32 = jax.lax.bitcast_convert_type(
        kmat.reshape(N, D_QK // 2, 2), jnp.int32)
    A = _sc_gather_dot(q, k32, nbr_flat)
    return _pool(A, value, Wout, bout.reshape(1, -1))


# speedup vs baseline: 1.3416x; 1.0092x over previous
import functools

import jax
import jax.numpy as jnp
import numpy as np
from jax import lax
from jax.experimental import pallas as pl
from jax.experimental.pallas import tpu as pltpu
from jax.experimental.pallas import tpu_sc as plsc

N = 4096
D_IN = 1024
D_H = 512
D_QK = 256
NNB = 9

NC = 2
NS = 16
LANES = 16
NW = NC * NS
ROWS_PER_W = N // NW
CHUNK = 16
NCHUNKS = ROWS_PER_W // CHUNK
NPAIR = D_QK // (2 * LANES)

_PERM = np.arange(D_QK).reshape(NPAIR, LANES, 2).transpose(0, 2, 1).reshape(-1)


def _proj_body(x_ref, w1_ref, b1_ref, wq_ref, wk_ref, wv_ref, bv_ref,
               q_ref, k_ref, v_ref):
    bf = jnp.bfloat16
    dense = jnp.dot(x_ref[...].astype(bf), w1_ref[...].astype(bf),
                    preferred_element_type=jnp.float32)
    dense = jnp.maximum(dense + b1_ref[...], 0.0).astype(bf)
    q_ref[...] = jnp.dot(dense, wq_ref[...].astype(bf),
                         preferred_element_type=jnp.float32) * (1.0 / 16.0)
    k_ref[...] = jnp.dot(dense, wk_ref[...].astype(bf),
                         preferred_element_type=jnp.float32).astype(bf)
    v_ref[...] = jnp.dot(dense, wv_ref[...].astype(bf),
                         preferred_element_type=jnp.float32) + bv_ref[...]


_PROJ_TILE = 512


def _projections(x, W1, b1, Wq, Wk, Wv, bv):
    grid = (N // _PROJ_TILE,)
    out_shape = [
        jax.ShapeDtypeStruct((N, D_QK), jnp.float32),
        jax.ShapeDtypeStruct((N, D_QK), jnp.bfloat16),
        jax.ShapeDtypeStruct((N, D_QK), jnp.float32),
    ]
    return pl.pallas_call(
        _proj_body,
        grid=grid,
        in_specs=[
            pl.BlockSpec((_PROJ_TILE, D_IN), lambda i: (i, 0)),
            pl.BlockSpec((D_IN, D_H), lambda i: (0, 0)),
            pl.BlockSpec((1, D_H), lambda i: (0, 0)),
            pl.BlockSpec((D_H, D_QK), lambda i: (0, 0)),
            pl.BlockSpec((D_H, D_QK), lambda i: (0, 0)),
            pl.BlockSpec((D_H, D_QK), lambda i: (0, 0)),
            pl.BlockSpec((1, D_QK), lambda i: (0, 0)),
        ],
        out_specs=[pl.BlockSpec((_PROJ_TILE, D_QK), lambda i: (i, 0))] * 3,
        out_shape=out_shape,
    )(x, W1, b1, Wq, Wk, Wv, bv)


def _sc_body(q_hbm, k_hbm, idx_hbm, out_hbm, qc0_v, qc1_v, idx_v, g0_v, g1_v,
             res_v, sem0, sem1, qsem0, qsem1):
    wid = lax.axis_index("s") * NC + lax.axis_index("c")
    row0 = wid * ROWS_PER_W
    pltpu.sync_copy(idx_hbm.at[pl.ds(row0 * NNB, ROWS_PER_W * NNB)], idx_v)

    lane = jnp.arange(LANES, dtype=jnp.int32)
    bufs = (g0_v, g1_v)
    qbufs = (qc0_v, qc1_v)
    sems = (sem0, sem1)
    qsems = (qsem0, qsem1)

    def _gather(ci, buf, sem):
        return pltpu.make_async_copy(
            k_hbm.at[idx_v.at[pl.ds(ci * CHUNK * NNB, CHUNK * NNB)]],
            buf, sem)

    def _qcopy(ci, buf, sem):
        return pltpu.make_async_copy(
            q_hbm.at[pl.ds(row0 + ci * CHUNK, CHUNK)], buf, sem)

    def _fire(ci, b):
        _gather(ci, bufs[b], sems[b]).start()
        _qcopy(ci, qbufs[b], qsems[b]).start()

    _fire(0, 0)

    def compute_chunk(ci, g_v, q_v):
        def row_body(r, carry):
            v3, v5, v9 = carry
            rr = r
            g0 = r * NNB
            t3 = jnp.zeros((LANES,), jnp.float32)
            t5 = jnp.zeros((LANES,), jnp.float32)
            t9 = jnp.zeros((LANES,), jnp.float32)
            for c in range(NPAIR):
                sl = pl.ds(c * LANES, LANES)
                gld = lambda j: plsc.bitcast(g_v[g0 + j, sl], jnp.bfloat16)
                s = gld(0) + gld(1) + gld(2)
                a, b = plsc.unpack(s, format=plsc.PackFormat.INTERLEAVED)
                qa = q_v[rr, pl.ds(c * 2 * LANES, LANES)]
                qb = q_v[rr, pl.ds(c * 2 * LANES + LANES, LANES)]
                t3 = t3 + qa * a + qb * b
                s = s + gld(3) + gld(4)
                a, b = plsc.unpack(s, format=plsc.PackFormat.INTERLEAVED)
                t5 = t5 + qa * a + qb * b
                s = s + gld(5) + gld(6) + gld(7) + gld(8)
                a, b = plsc.unpack(s, format=plsc.PackFormat.INTERLEAVED)
                t9 = t9 + qa * a + qb * b
            m = lane == r
            v3 = jnp.where(m, jnp.sum(t3), v3)
            v5 = jnp.where(m, jnp.sum(t5), v5)
            v9 = jnp.where(m, jnp.sum(t9), v9)
            return v3, v5, v9

        zero = jnp.zeros((LANES,), jnp.float32)
        v3, v5, v9 = lax.fori_loop(0, CHUNK, row_body, (zero, zero, zero))
        res_v[0, pl.ds(ci * CHUNK, CHUNK)] = v3
        res_v[1, pl.ds(ci * CHUNK, CHUNK)] = v5
        res_v[2, pl.ds(ci * CHUNK, CHUNK)] = v9

    def super_body(s, _):
        for b in range(2):
            ci = 2 * s + b

            @pl.when(ci + 1 < NCHUNKS)
            def _():
                _fire(ci + 1, 1 - b)

            _gather(ci, bufs[b], sems[b]).wait()
            _qcopy(ci, qbufs[b], qsems[b]).wait()
            compute_chunk(ci, bufs[b], qbufs[b])
        return 0

    lax.fori_loop(0, NCHUNKS // 2, super_body, 0)
    pltpu.sync_copy(res_v, out_hbm.at[:, pl.ds(row0, ROWS_PER_W)])


def _sc_gather_dot(q, kmat, nbr_flat):
    mesh = plsc.VectorSubcoreMesh(core_axis_name="c", subcore_axis_name="s",
                                  num_cores=NC, num_subcores=NS)
    kern = functools.partial(
        pl.kernel,
        out_type=jax.ShapeDtypeStruct((3, N), jnp.float32),
        mesh=mesh,
        compiler_params=pltpu.CompilerParams(needs_layout_passes=False),
        scratch_types=[
            pltpu.VMEM((CHUNK, D_QK), jnp.float32),
            pltpu.VMEM((CHUNK, D_QK), jnp.float32),
            pltpu.VMEM((ROWS_PER_W * NNB,), jnp.int32),
            pltpu.VMEM((CHUNK * NNB, D_QK // 2), jnp.int32),
            pltpu.VMEM((CHUNK * NNB, D_QK // 2), jnp.int32),
            pltpu.VMEM((3, ROWS_PER_W), jnp.float32),
            pltpu.SemaphoreType.DMA,
            pltpu.SemaphoreType.DMA,
            pltpu.SemaphoreType.DMA,
            pltpu.SemaphoreType.DMA,
        ],
    )(_sc_body)
    return kern(q, kmat, nbr_flat)


def _pool_body(a_ref, v_ref, wout_ref, bout_ref, o_ref):
    A = a_ref[...]
    m = jnp.max(A, axis=1, keepdims=True)
    e = jnp.exp(A - m)
    alpha = e / jnp.sum(e, axis=1, keepdims=True)
    s3 = jnp.dot(alpha, v_ref[...], preferred_element_type=jnp.float32)
    s = jnp.sum(s3, axis=0, keepdims=True)
    logits = jnp.dot(s, wout_ref[...],
                     preferred_element_type=jnp.float32) + bout_ref[...]
    ls = logits - jnp.max(logits, axis=1, keepdims=True)
    el = jnp.exp(ls)
    o_ref[...] = el / jnp.sum(el, axis=1, keepdims=True)


def _pool(A, value, Wout, bout):
    return pl.pallas_call(
        _pool_body,
        out_shape=jax.ShapeDtypeStruct((1, 2), jnp.float32),
    )(A, value, Wout, bout)


def kernel(x, neighbor_idx, W1, b1, Wq, Wk, Wv, bv, Wout, bout):
    q, kmat, value = _projections(x, W1, b1.reshape(1, -1), Wq[:, _PERM],
                                  Wk, Wv, bv.reshape(1, -1))
    nbr_flat = neighbor_idx.astype(jnp.int32).reshape(-1)
    k32 = jax.lax.bitcast_convert_type(
        kmat.reshape(N, D_QK // 2, 2), jnp.int32)
    A = _sc_gather_dot(q, k32, nbr_flat)
    return _pool(A, value, Wout, bout.reshape(1, -1))
